# Initial kernel scaffold; baseline (speedup 1.0000x reference)
import functools
import math

import jax
import jax.numpy as jnp
from jax import lax
from jax.experimental import pallas as pl
from jax.experimental.pallas import tpu as pltpu
from jax.experimental.pallas import tpu_sc as plsc

T, D, E, H = 2048, 768, 8, 3072
BT = 128
BH = 512
NH = H // BH
CAP = T + E * BT
NB = CAP // BT

_SC_INFO = plsc.get_sparse_core_info()
_NW = _SC_INFO.num_cores * _SC_INFO.num_subcores


def _gating_body(x_ref, wg_ref, bg_ref, top1_ref):
    logits = jnp.dot(x_ref[...], wg_ref[...], preferred_element_type=jnp.float32)
    logits = logits + bg_ref[...]
    m = jnp.max(logits, axis=1, keepdims=True)
    lane = lax.broadcasted_iota(jnp.int32, logits.shape, 1)
    cand = jnp.where(logits == m, lane, jnp.int32(2**30))
    top1_ref[...] = jnp.min(cand, axis=1, keepdims=True)


def _gating(x2, Wg, bg):
    return pl.pallas_call(
        _gating_body,
        out_shape=jax.ShapeDtypeStruct((T, 1), jnp.int32),
    )(x2, Wg, bg.reshape(1, E))


def _routing_meta(top1):
    onehot = (top1[:, None] == jnp.arange(E, dtype=jnp.int32)[None, :]).astype(jnp.int32)
    cum = jnp.cumsum(onehot, axis=0)
    counts = cum[-1]
    rank = jnp.take_along_axis(cum, top1[:, None], axis=1)[:, 0] - 1
    padded = ((counts + BT - 1) // BT) * BT
    pstart = jnp.concatenate(
        [jnp.zeros((1,), jnp.int32), jnp.cumsum(padded)]).astype(jnp.int32)
    pos = pstart[top1] + rank
    gidx = jnp.zeros((CAP,), jnp.int32).at[pos].set(
        jnp.arange(T, dtype=jnp.int32))
    block_expert = jnp.clip(
        jnp.searchsorted(pstart[1:], jnp.arange(NB, dtype=jnp.int32) * BT,
                         side="right"),
        0, E - 1).astype(jnp.int32)
    return gidx, pos, block_expert


def _sc_row_gather(table, idx, n_out):
    n_per_w = n_out // _NW
    mesh = plsc.VectorSubcoreMesh(core_axis_name="c", subcore_axis_name="s")

    @functools.partial(
        pl.kernel,
        out_type=jax.ShapeDtypeStruct((n_out, D), jnp.float32),
        mesh=mesh,
        scratch_types=[
            pltpu.VMEM((n_per_w,), jnp.int32),
            pltpu.VMEM((n_per_w, D), jnp.float32),
            pltpu.SemaphoreType.DMA,
        ],
    )
    def gather_kernel(table_hbm, idx_hbm, out_hbm, idx_v, rows_v, sem):
        wid = lax.axis_index("s") * _SC_INFO.num_cores + lax.axis_index("c")
        base = wid * n_per_w
        pltpu.sync_copy(idx_hbm.at[pl.ds(base, n_per_w)], idx_v)
        pltpu.async_copy(table_hbm.at[idx_v], rows_v, sem).wait()
        pltpu.sync_copy(rows_v, out_hbm.at[pl.ds(base, n_per_w)])

    return gather_kernel(table, idx)


def _mlp_body(be_ref, xs_ref, w1_ref, b1_ref, w2_ref, b2_ref, ys_ref):
    j = pl.program_id(0)
    b = pl.program_id(1)
    rows = pl.ds(b * BT, BT)
    x_blk = xs_ref[rows, :]
    h = jnp.dot(x_blk, w1_ref[0], preferred_element_type=jnp.float32)
    h = h + b1_ref[0][None, :]
    h = 0.5 * h * (1.0 + lax.erf(h * (1.0 / math.sqrt(2.0))))
    contrib = jnp.dot(h, w2_ref[0], preferred_element_type=jnp.float32)

    @pl.when(j == 0)
    def _():
        ys_ref[rows, :] = contrib + b2_ref[0][None, :]

    @pl.when(j > 0)
    def _():
        ys_ref[rows, :] += contrib


def _grouped_mlp(xs, W1, b1, W2, b2, block_expert):
    grid_spec = pltpu.PrefetchScalarGridSpec(
        num_scalar_prefetch=1,
        grid=(NH, NB),
        in_specs=[
            pl.BlockSpec((CAP, D), lambda j, b, be: (0, 0)),
            pl.BlockSpec((1, D, BH), lambda j, b, be: (be[b], 0, j)),
            pl.BlockSpec((1, BH), lambda j, b, be: (be[b], j)),
            pl.BlockSpec((1, BH, D), lambda j, b, be: (be[b], j, 0)),
            pl.BlockSpec((1, D), lambda j, b, be: (be[b], 0)),
        ],
        out_specs=pl.BlockSpec((CAP, D), lambda j, b, be: (0, 0)),
    )
    return pl.pallas_call(
        _mlp_body,
        grid_spec=grid_spec,
        out_shape=jax.ShapeDtypeStruct((CAP, D), jnp.float32),
        compiler_params=pltpu.CompilerParams(
            dimension_semantics=("arbitrary", "arbitrary")),
    )(block_expert, xs, W1, b1, W2, b2)


def kernel(x, Wg, bg, W1, b1, W2, b2):
    b, s, d = x.shape
    x2 = x.reshape(T, D)
    top1 = _gating(x2, Wg, bg)[:, 0]
    gidx, pos, block_expert = _routing_meta(top1)
    xs = _sc_row_gather(x2, gidx, CAP)
    ys = _grouped_mlp(xs, W1, b1, W2, b2, block_expert)
    out = _sc_row_gather(ys, pos, T)
    return out.reshape(b, s, d)


# speedup vs baseline: 2.3304x; 2.3304x over previous
import functools
import math

import jax
import jax.numpy as jnp
from jax import lax
from jax.experimental import pallas as pl
from jax.experimental.pallas import tpu as pltpu
from jax.experimental.pallas import tpu_sc as plsc

T, D, E, H = 2048, 768, 8, 3072
BT = 128
BH = 512
NH = H // BH
CAP = T + E * BT
NB = CAP // BT

_NC = 2
_NS = 16
_NW = _NC * _NS


def _gating_body(x_ref, wg_ref, bg_ref, top1_ref):
    logits = jnp.dot(x_ref[...], wg_ref[...], preferred_element_type=jnp.float32)
    logits = logits + bg_ref[...]
    m = jnp.max(logits, axis=1, keepdims=True)
    lane = lax.broadcasted_iota(jnp.int32, logits.shape, 1)
    cand = jnp.where(logits == m, lane, jnp.int32(2**30))
    top1_ref[...] = jnp.min(cand, axis=1, keepdims=True)


def _gating(x2, Wg, bg):
    return pl.pallas_call(
        _gating_body,
        out_shape=jax.ShapeDtypeStruct((T, 1), jnp.int32),
    )(x2, Wg, bg.reshape(1, E))


def _routing_meta(top1):
    onehot = (top1[:, None] == jnp.arange(E, dtype=jnp.int32)[None, :]).astype(jnp.int32)
    cum = jnp.cumsum(onehot, axis=0)
    counts = cum[-1]
    rank = jnp.take_along_axis(cum, top1[:, None], axis=1)[:, 0] - 1
    padded = ((counts + BT - 1) // BT) * BT
    pstart = jnp.concatenate(
        [jnp.zeros((1,), jnp.int32), jnp.cumsum(padded)]).astype(jnp.int32)
    pos = pstart[top1] + rank
    gidx = jnp.zeros((CAP,), jnp.int32).at[pos].set(
        jnp.arange(T, dtype=jnp.int32))
    block_expert = jnp.clip(
        jnp.searchsorted(pstart[1:], jnp.arange(NB, dtype=jnp.int32) * BT,
                         side="right"),
        0, E - 1).astype(jnp.int32)
    return gidx, pos, block_expert


def _sc_row_gather(table, idx, n_out):
    n_per_w = n_out // _NW
    mesh = plsc.VectorSubcoreMesh(core_axis_name="c", subcore_axis_name="s")

    @functools.partial(
        pl.kernel,
        out_type=jax.ShapeDtypeStruct((n_out, D), jnp.float32),
        mesh=mesh,
        scratch_types=[
            pltpu.VMEM((n_per_w,), jnp.int32),
            pltpu.VMEM((n_per_w, D), jnp.float32),
            pltpu.SemaphoreType.DMA,
        ],
    )
    def gather_kernel(table_hbm, idx_hbm, out_hbm, idx_v, rows_v, sem):
        wid = lax.axis_index("s") * _NC + lax.axis_index("c")
        base = wid * n_per_w
        pltpu.sync_copy(idx_hbm.at[pl.ds(base, n_per_w)], idx_v)
        pltpu.async_copy(table_hbm.at[idx_v], rows_v, sem).wait()
        pltpu.sync_copy(rows_v, out_hbm.at[pl.ds(base, n_per_w)])

    return gather_kernel(table, idx)


def _mlp_body(be_ref, xs_ref, w1_ref, b1_ref, w2_ref, b2_ref, ys_ref):
    j = pl.program_id(0)
    b = pl.program_id(1)
    rows = pl.ds(b * BT, BT)
    x_blk = xs_ref[rows, :]
    h = jnp.dot(x_blk, w1_ref[0], preferred_element_type=jnp.float32)
    h = h + b1_ref[0]
    h = 0.5 * h * (1.0 + lax.erf(h * (1.0 / math.sqrt(2.0))))
    contrib = jnp.dot(h, w2_ref[0], preferred_element_type=jnp.float32)

    @pl.when(j == 0)
    def _():
        ys_ref[rows, :] = contrib + b2_ref[0]

    @pl.when(j > 0)
    def _():
        ys_ref[rows, :] += contrib


def _grouped_mlp(xs, W1, b1, W2, b2, block_expert):
    grid_spec = pltpu.PrefetchScalarGridSpec(
        num_scalar_prefetch=1,
        grid=(NH, NB),
        in_specs=[
            pl.BlockSpec((CAP, D), lambda j, b, be: (0, 0)),
            pl.BlockSpec((1, D, BH), lambda j, b, be: (be[b], 0, j)),
            pl.BlockSpec((1, 1, BH), lambda j, b, be: (be[b], 0, j)),
            pl.BlockSpec((1, BH, D), lambda j, b, be: (be[b], j, 0)),
            pl.BlockSpec((1, 1, D), lambda j, b, be: (be[b], 0, 0)),
        ],
        out_specs=pl.BlockSpec((CAP, D), lambda j, b, be: (0, 0)),
    )
    return pl.pallas_call(
        _mlp_body,
        grid_spec=grid_spec,
        out_shape=jax.ShapeDtypeStruct((CAP, D), jnp.float32),
        compiler_params=pltpu.CompilerParams(
            dimension_semantics=("arbitrary", "arbitrary")),
    )(block_expert, xs, W1, b1.reshape(E, 1, H), W2, b2.reshape(E, 1, D))


def kernel(x, Wg, bg, W1, b1, W2, b2):
    b, s, d = x.shape
    x2 = x.reshape(T, D)
    top1 = _gating(x2, Wg, bg)[:, 0]
    gidx, pos, block_expert = _routing_meta(top1)
    xs = _sc_row_gather(x2, gidx, CAP)
    ys = _grouped_mlp(xs, W1, b1, W2, b2, block_expert)
    out = _sc_row_gather(ys, pos, T)
    return out.reshape(b, s, d)


# speedup vs baseline: 2.8434x; 1.2201x over previous
import functools
import math

import jax
import jax.numpy as jnp
from jax import lax
from jax.experimental import pallas as pl
from jax.experimental.pallas import tpu as pltpu
from jax.experimental.pallas import tpu_sc as plsc

T, D, E, H = 2048, 768, 8, 3072
BT = 128
BH = 512
NH = H // BH
CAP = T + E * BT
NB = CAP // BT

_NC = 2
_NS = 16
_NW = _NC * _NS


def _gating_body(x_ref, wg_ref, bg_ref, top1_ref):
    logits = jnp.dot(x_ref[...], wg_ref[...], preferred_element_type=jnp.float32)
    logits = logits + bg_ref[...]
    m = jnp.max(logits, axis=1, keepdims=True)
    lane = lax.broadcasted_iota(jnp.int32, logits.shape, 1)
    cand = jnp.where(logits == m, lane, jnp.int32(2**30))
    top1_ref[...] = jnp.min(cand, axis=1, keepdims=True)


def _gating(x2, Wg, bg):
    return pl.pallas_call(
        _gating_body,
        out_shape=jax.ShapeDtypeStruct((T, 1), jnp.int32),
    )(x2, Wg, bg.reshape(1, E))


def _routing_meta(top1):
    onehot = (top1[:, None] == jnp.arange(E, dtype=jnp.int32)[None, :]).astype(jnp.int32)
    cum = jnp.cumsum(onehot, axis=0)
    counts = cum[-1]
    rank = jnp.take_along_axis(cum, top1[:, None], axis=1)[:, 0] - 1
    padded = ((counts + BT - 1) // BT) * BT
    pstart = jnp.concatenate(
        [jnp.zeros((1,), jnp.int32), jnp.cumsum(padded)]).astype(jnp.int32)
    pos = pstart[top1] + rank
    gidx = (jnp.arange(CAP, dtype=jnp.int32) % T).at[pos].set(
        jnp.arange(T, dtype=jnp.int32))
    block_expert = jnp.clip(
        jnp.searchsorted(pstart[1:], jnp.arange(NB, dtype=jnp.int32) * BT,
                         side="right"),
        0, E - 1).astype(jnp.int32)
    return gidx, pos, block_expert


def _sc_row_gather(table, idx, n_out):
    n_per_w = n_out // _NW
    mesh = plsc.VectorSubcoreMesh(core_axis_name="c", subcore_axis_name="s")

    @functools.partial(
        pl.kernel,
        out_type=jax.ShapeDtypeStruct((n_out, D), jnp.float32),
        mesh=mesh,
        scratch_types=[
            pltpu.VMEM((n_per_w,), jnp.int32),
            pltpu.VMEM((n_per_w, D), jnp.float32),
            pltpu.SemaphoreType.DMA,
        ],
    )
    def gather_kernel(table_hbm, idx_hbm, out_hbm, idx_v, rows_v, sem):
        wid = lax.axis_index("s") * _NC + lax.axis_index("c")
        base = wid * n_per_w
        pltpu.sync_copy(idx_hbm.at[pl.ds(base, n_per_w)], idx_v)
        pltpu.async_copy(table_hbm.at[idx_v], rows_v, sem).wait()
        pltpu.sync_copy(rows_v, out_hbm.at[pl.ds(base, n_per_w)])

    return gather_kernel(table, idx)


def _mlp_body(be_ref, xs_ref, w1_ref, b1_ref, w2_ref, b2_ref, ys_ref):
    j = pl.program_id(0)
    b = pl.program_id(1)
    rows = pl.ds(b * BT, BT)
    x_blk = xs_ref[rows, :]
    h = jnp.dot(x_blk, w1_ref[0], preferred_element_type=jnp.float32)
    h = h + b1_ref[0]
    h = 0.5 * h * (1.0 + lax.erf(h * (1.0 / math.sqrt(2.0))))
    contrib = jnp.dot(h, w2_ref[0], preferred_element_type=jnp.float32)

    @pl.when(j == 0)
    def _():
        ys_ref[rows, :] = contrib + b2_ref[0]

    @pl.when(j > 0)
    def _():
        ys_ref[rows, :] += contrib


def _grouped_mlp(xs, W1, b1, W2, b2, block_expert):
    grid_spec = pltpu.PrefetchScalarGridSpec(
        num_scalar_prefetch=1,
        grid=(NH, NB),
        in_specs=[
            pl.BlockSpec((CAP, D), lambda j, b, be: (0, 0)),
            pl.BlockSpec((1, D, BH), lambda j, b, be: (be[b], 0, j)),
            pl.BlockSpec((1, 1, BH), lambda j, b, be: (be[b], 0, j)),
            pl.BlockSpec((1, BH, D), lambda j, b, be: (be[b], j, 0)),
            pl.BlockSpec((1, 1, D), lambda j, b, be: (be[b], 0, 0)),
        ],
        out_specs=pl.BlockSpec((CAP, D), lambda j, b, be: (0, 0)),
    )
    return pl.pallas_call(
        _mlp_body,
        grid_spec=grid_spec,
        out_shape=jax.ShapeDtypeStruct((CAP, D), jnp.float32),
        compiler_params=pltpu.CompilerParams(
            dimension_semantics=("arbitrary", "arbitrary")),
    )(block_expert, xs, W1, b1.reshape(E, 1, H), W2, b2.reshape(E, 1, D))


def kernel(x, Wg, bg, W1, b1, W2, b2):
    b, s, d = x.shape
    x2 = x.reshape(T, D)
    top1 = _gating(x2, Wg, bg)[:, 0]
    gidx, pos, block_expert = _routing_meta(top1)
    xs = _sc_row_gather(x2, gidx, CAP)
    ys = _grouped_mlp(xs, W1, b1, W2, b2, block_expert)
    out = _sc_row_gather(ys, pos, T)
    return out.reshape(b, s, d)


# speedup vs baseline: 3.2886x; 1.1566x over previous
import functools
import math

import jax
import jax.numpy as jnp
from jax import lax
from jax.experimental import pallas as pl
from jax.experimental.pallas import tpu as pltpu
from jax.experimental.pallas import tpu_sc as plsc

T, D, E, H = 2048, 768, 8, 3072
BT = 128
BH = 768
NH = H // BH
CAP = T + E * BT
NB = CAP // BT

_NC = 2
_NS = 16
_NW = _NC * _NS


def _gating_body(x_ref, wg_ref, bg_ref, top1_ref):
    logits = jnp.dot(x_ref[...], wg_ref[...], preferred_element_type=jnp.float32)
    logits = logits + bg_ref[...]
    m = jnp.max(logits, axis=1, keepdims=True)
    lane = lax.broadcasted_iota(jnp.int32, logits.shape, 1)
    cand = jnp.where(logits == m, lane, jnp.int32(2**30))
    top1_ref[...] = jnp.min(cand, axis=1, keepdims=True)


def _gating(x2, Wg, bg):
    return pl.pallas_call(
        _gating_body,
        out_shape=jax.ShapeDtypeStruct((T, 1), jnp.int32),
    )(x2, Wg, bg.reshape(1, E))


def _routing_meta(top1):
    onehot = (top1[:, None] == jnp.arange(E, dtype=jnp.int32)[None, :]).astype(jnp.int32)
    cum = jnp.cumsum(onehot, axis=0)
    counts = cum[-1]
    rank = jnp.take_along_axis(cum, top1[:, None], axis=1)[:, 0] - 1
    padded = ((counts + BT - 1) // BT) * BT
    pstart = jnp.concatenate(
        [jnp.zeros((1,), jnp.int32), jnp.cumsum(padded)]).astype(jnp.int32)
    pos = pstart[top1] + rank
    gidx = (jnp.arange(CAP, dtype=jnp.int32) % T).at[pos].set(
        jnp.arange(T, dtype=jnp.int32))
    block_expert = jnp.clip(
        jnp.searchsorted(pstart[1:], jnp.arange(NB, dtype=jnp.int32) * BT,
                         side="right"),
        0, E - 1).astype(jnp.int32)
    return gidx, pos, block_expert


def _sc_row_gather(table, idx, n_out):
    n_per_w = n_out // _NW
    mesh = plsc.VectorSubcoreMesh(core_axis_name="c", subcore_axis_name="s")

    @functools.partial(
        pl.kernel,
        out_type=jax.ShapeDtypeStruct((n_out, D), jnp.float32),
        mesh=mesh,
        scratch_types=[
            pltpu.VMEM((n_per_w,), jnp.int32),
            pltpu.VMEM((n_per_w, D), jnp.float32),
            pltpu.SemaphoreType.DMA,
        ],
    )
    def gather_kernel(table_hbm, idx_hbm, out_hbm, idx_v, rows_v, sem):
        wid = lax.axis_index("s") * _NC + lax.axis_index("c")
        base = wid * n_per_w
        pltpu.sync_copy(idx_hbm.at[pl.ds(base, n_per_w)], idx_v)
        pltpu.async_copy(table_hbm.at[idx_v], rows_v, sem).wait()
        pltpu.sync_copy(rows_v, out_hbm.at[pl.ds(base, n_per_w)])

    return gather_kernel(table, idx)


def _mlp_body(be_ref, xs_ref, w1_ref, b1_ref, w2_ref, b2_ref, ys_ref):
    j = pl.program_id(0)
    b = pl.program_id(1)
    rows = pl.ds(b * BT, BT)
    x_blk = xs_ref[rows, :]
    h = jnp.dot(x_blk, w1_ref[0], preferred_element_type=jnp.float32)
    h = h + b1_ref[0]
    h = 0.5 * h * (1.0 + lax.erf(h * (1.0 / math.sqrt(2.0))))
    contrib = jnp.dot(h, w2_ref[0], preferred_element_type=jnp.float32)

    @pl.when(j == 0)
    def _():
        ys_ref[rows, :] = contrib + b2_ref[0]

    @pl.when(j > 0)
    def _():
        ys_ref[rows, :] += contrib


def _grouped_mlp(xs, W1, b1, W2, b2, block_expert):
    grid_spec = pltpu.PrefetchScalarGridSpec(
        num_scalar_prefetch=1,
        grid=(NH, NB),
        in_specs=[
            pl.BlockSpec((CAP, D), lambda j, b, be: (0, 0)),
            pl.BlockSpec((1, D, BH), lambda j, b, be: (be[b], 0, j)),
            pl.BlockSpec((1, 1, BH), lambda j, b, be: (be[b], 0, j)),
            pl.BlockSpec((1, BH, D), lambda j, b, be: (be[b], j, 0)),
            pl.BlockSpec((1, 1, D), lambda j, b, be: (be[b], 0, 0)),
        ],
        out_specs=pl.BlockSpec((CAP, D), lambda j, b, be: (0, 0)),
    )
    return pl.pallas_call(
        _mlp_body,
        grid_spec=grid_spec,
        out_shape=jax.ShapeDtypeStruct((CAP, D), jnp.float32),
        compiler_params=pltpu.CompilerParams(
            dimension_semantics=("arbitrary", "arbitrary")),
    )(block_expert, xs, W1, b1.reshape(E, 1, H), W2, b2.reshape(E, 1, D))


def kernel(x, Wg, bg, W1, b1, W2, b2):
    b, s, d = x.shape
    x2 = x.reshape(T, D)
    top1 = _gating(x2, Wg, bg)[:, 0]
    gidx, pos, block_expert = _routing_meta(top1)
    xs = _sc_row_gather(x2, gidx, CAP)
    ys = _grouped_mlp(xs, W1, b1, W2, b2, block_expert)
    out = _sc_row_gather(ys, pos, T)
    return out.reshape(b, s, d)


# speedup vs baseline: 3.5866x; 1.0906x over previous
import functools
import math

import jax
import jax.numpy as jnp
from jax import lax
from jax.experimental import pallas as pl
from jax.experimental.pallas import tpu as pltpu
from jax.experimental.pallas import tpu_sc as plsc

T, D, E, H = 2048, 768, 8, 3072
BT = 128
BH = 1024
NH = H // BH
CAP = T + E * BT
NB = CAP // BT

_NC = 2
_NS = 16
_NW = _NC * _NS


def _gating_body(x_ref, wg_ref, bg_ref, top1_ref):
    logits = jnp.dot(x_ref[...], wg_ref[...], preferred_element_type=jnp.float32)
    logits = logits + bg_ref[...]
    m = jnp.max(logits, axis=1, keepdims=True)
    lane = lax.broadcasted_iota(jnp.int32, logits.shape, 1)
    cand = jnp.where(logits == m, lane, jnp.int32(2**30))
    top1_ref[...] = jnp.min(cand, axis=1, keepdims=True)


def _gating(x2, Wg, bg):
    return pl.pallas_call(
        _gating_body,
        out_shape=jax.ShapeDtypeStruct((T, 1), jnp.int32),
    )(x2, Wg, bg.reshape(1, E))


def _routing_meta(top1):
    onehot = (top1[:, None] == jnp.arange(E, dtype=jnp.int32)[None, :]).astype(jnp.int32)
    cum = jnp.cumsum(onehot, axis=0)
    counts = cum[-1]
    rank = jnp.take_along_axis(cum, top1[:, None], axis=1)[:, 0] - 1
    padded = ((counts + BT - 1) // BT) * BT
    pstart = jnp.concatenate(
        [jnp.zeros((1,), jnp.int32), jnp.cumsum(padded)]).astype(jnp.int32)
    pos = pstart[top1] + rank
    gidx = (jnp.arange(CAP, dtype=jnp.int32) % T).at[pos].set(
        jnp.arange(T, dtype=jnp.int32))
    block_expert = jnp.clip(
        jnp.searchsorted(pstart[1:], jnp.arange(NB, dtype=jnp.int32) * BT,
                         side="right"),
        0, E - 1).astype(jnp.int32)
    return gidx, pos, block_expert


def _sc_row_gather(table, idx, n_out):
    n_per_w = n_out // _NW
    mesh = plsc.VectorSubcoreMesh(core_axis_name="c", subcore_axis_name="s")

    @functools.partial(
        pl.kernel,
        out_type=jax.ShapeDtypeStruct((n_out, D), jnp.float32),
        mesh=mesh,
        scratch_types=[
            pltpu.VMEM((n_per_w,), jnp.int32),
            pltpu.VMEM((n_per_w, D), jnp.float32),
            pltpu.SemaphoreType.DMA,
        ],
    )
    def gather_kernel(table_hbm, idx_hbm, out_hbm, idx_v, rows_v, sem):
        wid = lax.axis_index("s") * _NC + lax.axis_index("c")
        base = wid * n_per_w
        pltpu.sync_copy(idx_hbm.at[pl.ds(base, n_per_w)], idx_v)
        pltpu.async_copy(table_hbm.at[idx_v], rows_v, sem).wait()
        pltpu.sync_copy(rows_v, out_hbm.at[pl.ds(base, n_per_w)])

    return gather_kernel(table, idx)


def _mlp_body(be_ref, xs_ref, w1_ref, b1_ref, w2_ref, b2_ref, ys_ref):
    j = pl.program_id(0)
    b = pl.program_id(1)
    rows = pl.ds(b * BT, BT)
    x_blk = xs_ref[rows, :]
    h = jnp.dot(x_blk, w1_ref[0], preferred_element_type=jnp.float32)
    h = h + b1_ref[0]
    h = 0.5 * h * (1.0 + lax.erf(h * (1.0 / math.sqrt(2.0))))
    contrib = jnp.dot(h, w2_ref[0], preferred_element_type=jnp.float32)

    @pl.when(j == 0)
    def _():
        ys_ref[rows, :] = contrib + b2_ref[0]

    @pl.when(j > 0)
    def _():
        ys_ref[rows, :] += contrib


def _grouped_mlp(xs, W1, b1, W2, b2, block_expert):
    grid_spec = pltpu.PrefetchScalarGridSpec(
        num_scalar_prefetch=1,
        grid=(NH, NB),
        in_specs=[
            pl.BlockSpec((CAP, D), lambda j, b, be: (0, 0)),
            pl.BlockSpec((1, D, BH), lambda j, b, be: (be[b], 0, j)),
            pl.BlockSpec((1, 1, BH), lambda j, b, be: (be[b], 0, j)),
            pl.BlockSpec((1, BH, D), lambda j, b, be: (be[b], j, 0)),
            pl.BlockSpec((1, 1, D), lambda j, b, be: (be[b], 0, 0)),
        ],
        out_specs=pl.BlockSpec((CAP, D), lambda j, b, be: (0, 0)),
    )
    return pl.pallas_call(
        _mlp_body,
        grid_spec=grid_spec,
        out_shape=jax.ShapeDtypeStruct((CAP, D), jnp.float32),
        compiler_params=pltpu.CompilerParams(
            dimension_semantics=("arbitrary", "arbitrary")),
    )(block_expert, xs, W1, b1.reshape(E, 1, H), W2, b2.reshape(E, 1, D))


def kernel(x, Wg, bg, W1, b1, W2, b2):
    b, s, d = x.shape
    x2 = x.reshape(T, D)
    top1 = _gating(x2, Wg, bg)[:, 0]
    gidx, pos, block_expert = _routing_meta(top1)
    xs = _sc_row_gather(x2, gidx, CAP)
    ys = _grouped_mlp(xs, W1, b1, W2, b2, block_expert)
    out = _sc_row_gather(ys, pos, T)
    return out.reshape(b, s, d)


# speedup vs baseline: 3.8431x; 1.0715x over previous
import functools
import math

import jax
import jax.numpy as jnp
from jax import lax
from jax.experimental import pallas as pl
from jax.experimental.pallas import tpu as pltpu
from jax.experimental.pallas import tpu_sc as plsc

T, D, E, H = 2048, 768, 8, 3072
BT = 128
BH = 1536
NH = H // BH
CAP = T + E * BT
NB = CAP // BT

_NC = 2
_NS = 16
_NW = _NC * _NS


def _gating_body(x_ref, wg_ref, bg_ref, top1_ref):
    logits = jnp.dot(x_ref[...], wg_ref[...], preferred_element_type=jnp.float32)
    logits = logits + bg_ref[...]
    m = jnp.max(logits, axis=1, keepdims=True)
    lane = lax.broadcasted_iota(jnp.int32, logits.shape, 1)
    cand = jnp.where(logits == m, lane, jnp.int32(2**30))
    top1_ref[...] = jnp.min(cand, axis=1, keepdims=True)


def _gating(x2, Wg, bg):
    return pl.pallas_call(
        _gating_body,
        out_shape=jax.ShapeDtypeStruct((T, 1), jnp.int32),
    )(x2, Wg, bg.reshape(1, E))


def _routing_meta(top1):
    onehot = (top1[:, None] == jnp.arange(E, dtype=jnp.int32)[None, :]).astype(jnp.int32)
    cum = jnp.cumsum(onehot, axis=0)
    counts = cum[-1]
    rank = jnp.take_along_axis(cum, top1[:, None], axis=1)[:, 0] - 1
    padded = ((counts + BT - 1) // BT) * BT
    pstart = jnp.concatenate(
        [jnp.zeros((1,), jnp.int32), jnp.cumsum(padded)]).astype(jnp.int32)
    pos = pstart[top1] + rank
    gidx = (jnp.arange(CAP, dtype=jnp.int32) % T).at[pos].set(
        jnp.arange(T, dtype=jnp.int32))
    block_expert = jnp.clip(
        jnp.searchsorted(pstart[1:], jnp.arange(NB, dtype=jnp.int32) * BT,
                         side="right"),
        0, E - 1).astype(jnp.int32)
    return gidx, pos, block_expert


def _sc_row_gather(table, idx, n_out):
    n_per_w = n_out // _NW
    mesh = plsc.VectorSubcoreMesh(core_axis_name="c", subcore_axis_name="s")

    @functools.partial(
        pl.kernel,
        out_type=jax.ShapeDtypeStruct((n_out, D), jnp.float32),
        mesh=mesh,
        scratch_types=[
            pltpu.VMEM((n_per_w,), jnp.int32),
            pltpu.VMEM((n_per_w, D), jnp.float32),
            pltpu.SemaphoreType.DMA,
        ],
    )
    def gather_kernel(table_hbm, idx_hbm, out_hbm, idx_v, rows_v, sem):
        wid = lax.axis_index("s") * _NC + lax.axis_index("c")
        base = wid * n_per_w
        pltpu.sync_copy(idx_hbm.at[pl.ds(base, n_per_w)], idx_v)
        pltpu.async_copy(table_hbm.at[idx_v], rows_v, sem).wait()
        pltpu.sync_copy(rows_v, out_hbm.at[pl.ds(base, n_per_w)])

    return gather_kernel(table, idx)


def _mlp_body(be_ref, xs_ref, w1_ref, b1_ref, w2_ref, b2_ref, ys_ref):
    j = pl.program_id(0)
    b = pl.program_id(1)
    rows = pl.ds(b * BT, BT)
    x_blk = xs_ref[rows, :]
    h = jnp.dot(x_blk, w1_ref[0], preferred_element_type=jnp.float32)
    h = h + b1_ref[0]
    h = 0.5 * h * (1.0 + lax.erf(h * (1.0 / math.sqrt(2.0))))
    contrib = jnp.dot(h, w2_ref[0], preferred_element_type=jnp.float32)

    @pl.when(j == 0)
    def _():
        ys_ref[rows, :] = contrib + b2_ref[0]

    @pl.when(j > 0)
    def _():
        ys_ref[rows, :] += contrib


def _grouped_mlp(xs, W1, b1, W2, b2, block_expert):
    grid_spec = pltpu.PrefetchScalarGridSpec(
        num_scalar_prefetch=1,
        grid=(NH, NB),
        in_specs=[
            pl.BlockSpec((CAP, D), lambda j, b, be: (0, 0)),
            pl.BlockSpec((1, D, BH), lambda j, b, be: (be[b], 0, j)),
            pl.BlockSpec((1, 1, BH), lambda j, b, be: (be[b], 0, j)),
            pl.BlockSpec((1, BH, D), lambda j, b, be: (be[b], j, 0)),
            pl.BlockSpec((1, 1, D), lambda j, b, be: (be[b], 0, 0)),
        ],
        out_specs=pl.BlockSpec((CAP, D), lambda j, b, be: (0, 0)),
    )
    return pl.pallas_call(
        _mlp_body,
        grid_spec=grid_spec,
        out_shape=jax.ShapeDtypeStruct((CAP, D), jnp.float32),
        compiler_params=pltpu.CompilerParams(
            dimension_semantics=("arbitrary", "arbitrary")),
    )(block_expert, xs, W1, b1.reshape(E, 1, H), W2, b2.reshape(E, 1, D))


def kernel(x, Wg, bg, W1, b1, W2, b2):
    b, s, d = x.shape
    x2 = x.reshape(T, D)
    top1 = _gating(x2, Wg, bg)[:, 0]
    gidx, pos, block_expert = _routing_meta(top1)
    xs = _sc_row_gather(x2, gidx, CAP)
    ys = _grouped_mlp(xs, W1, b1, W2, b2, block_expert)
    out = _sc_row_gather(ys, pos, T)
    return out.reshape(b, s, d)


# speedup vs baseline: 3.9708x; 1.0332x over previous
import functools
import math

import jax
import jax.numpy as jnp
from jax import lax
from jax.experimental import pallas as pl
from jax.experimental.pallas import tpu as pltpu
from jax.experimental.pallas import tpu_sc as plsc

T, D, E, H = 2048, 768, 8, 3072
BT = 256
BH = 1536
NH = H // BH
CAP = T + E * BT
NB = CAP // BT

_NC = 2
_NS = 16
_NW = _NC * _NS


def _gating_body(x_ref, wg_ref, bg_ref, top1_ref):
    logits = jnp.dot(x_ref[...], wg_ref[...], preferred_element_type=jnp.float32)
    logits = logits + bg_ref[...]
    m = jnp.max(logits, axis=1, keepdims=True)
    lane = lax.broadcasted_iota(jnp.int32, logits.shape, 1)
    cand = jnp.where(logits == m, lane, jnp.int32(2**30))
    top1_ref[...] = jnp.min(cand, axis=1, keepdims=True)


def _gating(x2, Wg, bg):
    return pl.pallas_call(
        _gating_body,
        out_shape=jax.ShapeDtypeStruct((T, 1), jnp.int32),
    )(x2, Wg, bg.reshape(1, E))


def _routing_meta(top1):
    onehot = (top1[:, None] == jnp.arange(E, dtype=jnp.int32)[None, :]).astype(jnp.int32)
    cum = jnp.cumsum(onehot, axis=0)
    counts = cum[-1]
    rank = jnp.take_along_axis(cum, top1[:, None], axis=1)[:, 0] - 1
    padded = ((counts + BT - 1) // BT) * BT
    pstart = jnp.concatenate(
        [jnp.zeros((1,), jnp.int32), jnp.cumsum(padded)]).astype(jnp.int32)
    pos = pstart[top1] + rank
    gidx = (jnp.arange(CAP, dtype=jnp.int32) % T).at[pos].set(
        jnp.arange(T, dtype=jnp.int32))
    block_expert = jnp.clip(
        jnp.searchsorted(pstart[1:], jnp.arange(NB, dtype=jnp.int32) * BT,
                         side="right"),
        0, E - 1).astype(jnp.int32)
    return gidx, pos, block_expert


def _sc_row_gather(table, idx, n_out):
    n_per_w = n_out // _NW
    mesh = plsc.VectorSubcoreMesh(core_axis_name="c", subcore_axis_name="s")

    @functools.partial(
        pl.kernel,
        out_type=jax.ShapeDtypeStruct((n_out, D), jnp.float32),
        mesh=mesh,
        scratch_types=[
            pltpu.VMEM((n_per_w,), jnp.int32),
            pltpu.VMEM((n_per_w, D), jnp.float32),
            pltpu.SemaphoreType.DMA,
        ],
    )
    def gather_kernel(table_hbm, idx_hbm, out_hbm, idx_v, rows_v, sem):
        wid = lax.axis_index("s") * _NC + lax.axis_index("c")
        base = wid * n_per_w
        pltpu.sync_copy(idx_hbm.at[pl.ds(base, n_per_w)], idx_v)
        pltpu.async_copy(table_hbm.at[idx_v], rows_v, sem).wait()
        pltpu.sync_copy(rows_v, out_hbm.at[pl.ds(base, n_per_w)])

    return gather_kernel(table, idx)


def _mlp_body(be_ref, xs_ref, w1_ref, b1_ref, w2_ref, b2_ref, ys_ref):
    j = pl.program_id(0)
    b = pl.program_id(1)
    rows = pl.ds(b * BT, BT)
    x_blk = xs_ref[rows, :]
    h = jnp.dot(x_blk, w1_ref[0], preferred_element_type=jnp.float32)
    h = h + b1_ref[0]
    h = 0.5 * h * (1.0 + lax.erf(h * (1.0 / math.sqrt(2.0))))
    contrib = jnp.dot(h, w2_ref[0], preferred_element_type=jnp.float32)

    @pl.when(j == 0)
    def _():
        ys_ref[rows, :] = contrib + b2_ref[0]

    @pl.when(j > 0)
    def _():
        ys_ref[rows, :] += contrib


def _grouped_mlp(xs, W1, b1, W2, b2, block_expert):
    grid_spec = pltpu.PrefetchScalarGridSpec(
        num_scalar_prefetch=1,
        grid=(NH, NB),
        in_specs=[
            pl.BlockSpec((CAP, D), lambda j, b, be: (0, 0)),
            pl.BlockSpec((1, D, BH), lambda j, b, be: (be[b], 0, j)),
            pl.BlockSpec((1, 1, BH), lambda j, b, be: (be[b], 0, j)),
            pl.BlockSpec((1, BH, D), lambda j, b, be: (be[b], j, 0)),
            pl.BlockSpec((1, 1, D), lambda j, b, be: (be[b], 0, 0)),
        ],
        out_specs=pl.BlockSpec((CAP, D), lambda j, b, be: (0, 0)),
    )
    return pl.pallas_call(
        _mlp_body,
        grid_spec=grid_spec,
        out_shape=jax.ShapeDtypeStruct((CAP, D), jnp.float32),
        compiler_params=pltpu.CompilerParams(
            dimension_semantics=("arbitrary", "arbitrary")),
    )(block_expert, xs, W1, b1.reshape(E, 1, H), W2, b2.reshape(E, 1, D))


def kernel(x, Wg, bg, W1, b1, W2, b2):
    b, s, d = x.shape
    x2 = x.reshape(T, D)
    top1 = _gating(x2, Wg, bg)[:, 0]
    gidx, pos, block_expert = _routing_meta(top1)
    xs = _sc_row_gather(x2, gidx, CAP)
    ys = _grouped_mlp(xs, W1, b1, W2, b2, block_expert)
    out = _sc_row_gather(ys, pos, T)
    return out.reshape(b, s, d)


# speedup vs baseline: 4.8348x; 1.2176x over previous
import functools
import math

import jax
import jax.numpy as jnp
from jax import lax
from jax.experimental import pallas as pl
from jax.experimental.pallas import tpu as pltpu
from jax.experimental.pallas import tpu_sc as plsc

T, D, E, H = 2048, 768, 8, 3072
BT = 256
BH = 1536
NH = H // BH
CAP = T + E * BT
NB = CAP // BT

_NC = 2
_NS = 16
_NW = _NC * _NS


_RC = 256


def _gating_body(x_ref, wg_ref, bg_ref, pos_ref, be_ref):
    logits = jnp.dot(x_ref[...], wg_ref[...], preferred_element_type=jnp.float32)
    logits = logits + bg_ref[...]
    m = jnp.max(logits, axis=1, keepdims=True)
    lane = lax.broadcasted_iota(jnp.int32, logits.shape, 1)
    cand = jnp.where(logits == m, lane, jnp.int32(2**30))
    top1 = jnp.min(cand, axis=1, keepdims=True)
    onehot = (lane == top1).astype(jnp.float32)

    col = lax.broadcasted_iota(jnp.int32, (_RC, T), 1)
    cum_chunks = []
    for k in range(T // _RC):
        row = lax.broadcasted_iota(jnp.int32, (_RC, T), 0) + (k * _RC)
        l_blk = jnp.where(col <= row, 1.0, 0.0)
        cum_chunks.append(jnp.dot(l_blk, onehot,
                                  preferred_element_type=jnp.float32))
    cum = jnp.concatenate(cum_chunks, axis=0)
    rank = jnp.sum(onehot * (cum - 1.0), axis=1, keepdims=True)

    counts = cum[T - 1:T, :]
    padded = jnp.floor((counts + (BT - 1)) / BT) * BT
    er = lax.broadcasted_iota(jnp.int32, (E, E), 0)
    ec = lax.broadcasted_iota(jnp.int32, (E, E), 1)
    strict = jnp.where(er < ec, 1.0, 0.0)
    pstart = jnp.dot(padded, strict, preferred_element_type=jnp.float32)
    pend = pstart + padded

    pos = jnp.sum(onehot * pstart, axis=1, keepdims=True) + rank
    pos_ref[...] = pos.astype(jnp.int32)

    kbt = (lax.broadcasted_iota(jnp.int32, (NB, E), 0) * BT).astype(jnp.float32)
    be = jnp.sum(jnp.where(jnp.broadcast_to(pend, (NB, E)) <= kbt, 1, 0),
                 axis=1, keepdims=True)
    be_ref[...] = jnp.minimum(be, E - 1).astype(jnp.int32)


def _gating_route(x2, Wg, bg):
    return pl.pallas_call(
        _gating_body,
        out_shape=(jax.ShapeDtypeStruct((T, 1), jnp.int32),
                   jax.ShapeDtypeStruct((NB, 1), jnp.int32)),
    )(x2, Wg, bg.reshape(1, E))


def _sc_dispatch(x2, pos):
    n_per_w = T // _NW
    mesh = plsc.VectorSubcoreMesh(core_axis_name="c", subcore_axis_name="s")

    @functools.partial(
        pl.kernel,
        out_type=jax.ShapeDtypeStruct((CAP, D), jnp.float32),
        mesh=mesh,
        scratch_types=[
            pltpu.VMEM((n_per_w,), jnp.int32),
            pltpu.VMEM((n_per_w, D), jnp.float32),
            pltpu.SemaphoreType.DMA,
        ],
    )
    def scatter_kernel(x_hbm, pos_hbm, out_hbm, idx_v, rows_v, sem):
        wid = lax.axis_index("s") * _NC + lax.axis_index("c")
        base = wid * n_per_w
        pltpu.sync_copy(pos_hbm.at[pl.ds(base, n_per_w)], idx_v)
        pltpu.sync_copy(x_hbm.at[pl.ds(base, n_per_w)], rows_v)
        pltpu.async_copy(rows_v, out_hbm.at[idx_v], sem).wait()

    return scatter_kernel(x2, pos)


def _sc_row_gather(table, idx, n_out):
    n_per_w = n_out // _NW
    mesh = plsc.VectorSubcoreMesh(core_axis_name="c", subcore_axis_name="s")

    @functools.partial(
        pl.kernel,
        out_type=jax.ShapeDtypeStruct((n_out, D), jnp.float32),
        mesh=mesh,
        scratch_types=[
            pltpu.VMEM((n_per_w,), jnp.int32),
            pltpu.VMEM((n_per_w, D), jnp.float32),
            pltpu.SemaphoreType.DMA,
        ],
    )
    def gather_kernel(table_hbm, idx_hbm, out_hbm, idx_v, rows_v, sem):
        wid = lax.axis_index("s") * _NC + lax.axis_index("c")
        base = wid * n_per_w
        pltpu.sync_copy(idx_hbm.at[pl.ds(base, n_per_w)], idx_v)
        pltpu.async_copy(table_hbm.at[idx_v], rows_v, sem).wait()
        pltpu.sync_copy(rows_v, out_hbm.at[pl.ds(base, n_per_w)])

    return gather_kernel(table, idx)


def _mlp_body(be_ref, xs_ref, w1_ref, b1_ref, w2_ref, b2_ref, ys_ref):
    j = pl.program_id(0)
    b = pl.program_id(1)
    rows = pl.ds(b * BT, BT)
    x_blk = xs_ref[rows, :]
    h = jnp.dot(x_blk, w1_ref[0], preferred_element_type=jnp.float32)
    h = h + b1_ref[0]
    h = 0.5 * h * (1.0 + lax.erf(h * (1.0 / math.sqrt(2.0))))
    contrib = jnp.dot(h, w2_ref[0], preferred_element_type=jnp.float32)

    @pl.when(j == 0)
    def _():
        ys_ref[rows, :] = contrib + b2_ref[0]

    @pl.when(j > 0)
    def _():
        ys_ref[rows, :] += contrib


def _grouped_mlp(xs, W1, b1, W2, b2, block_expert):
    grid_spec = pltpu.PrefetchScalarGridSpec(
        num_scalar_prefetch=1,
        grid=(NH, NB),
        in_specs=[
            pl.BlockSpec((CAP, D), lambda j, b, be: (0, 0)),
            pl.BlockSpec((1, D, BH), lambda j, b, be: (be[b], 0, j)),
            pl.BlockSpec((1, 1, BH), lambda j, b, be: (be[b], 0, j)),
            pl.BlockSpec((1, BH, D), lambda j, b, be: (be[b], j, 0)),
            pl.BlockSpec((1, 1, D), lambda j, b, be: (be[b], 0, 0)),
        ],
        out_specs=pl.BlockSpec((CAP, D), lambda j, b, be: (0, 0)),
    )
    return pl.pallas_call(
        _mlp_body,
        grid_spec=grid_spec,
        out_shape=jax.ShapeDtypeStruct((CAP, D), jnp.float32),
        compiler_params=pltpu.CompilerParams(
            dimension_semantics=("arbitrary", "arbitrary")),
    )(block_expert, xs, W1, b1.reshape(E, 1, H), W2, b2.reshape(E, 1, D))


def kernel(x, Wg, bg, W1, b1, W2, b2):
    b, s, d = x.shape
    x2 = x.reshape(T, D)
    pos, block_expert = _gating_route(x2, Wg, bg)
    pos = pos.reshape(T)
    xs = _sc_dispatch(x2, pos)
    ys = _grouped_mlp(xs, W1, b1, W2, b2, block_expert.reshape(NB))
    out = _sc_row_gather(ys, pos, T)
    return out.reshape(b, s, d)
